# speedup vs baseline: 13.6867x; 13.6867x over previous
import functools

import jax
import jax.numpy as jnp
from jax import lax
from jax.experimental import pallas as pl
from jax.experimental.pallas import tpu as pltpu
from jax.experimental.pallas import tpu_sc as plsc

NC = 2
NS = 16
NW = NC * NS
CH = 128
BLK = 1280


def _sc_mesh():
  return plsc.VectorSubcoreMesh(
      core_axis_name="c", subcore_axis_name="s",
      num_cores=NC, num_subcores=NS)


def _make_deg_kernel(n_pad, e_pad):
  per_w = e_pad // NW
  n_chunks = per_w // CH
  rows_per_s = n_pad // NS

  @functools.partial(
      pl.kernel,
      out_type=(jax.ShapeDtypeStruct((n_pad,), jnp.float32),
                jax.ShapeDtypeStruct((n_pad,), jnp.float32)),
      mesh=_sc_mesh(),
      scratch_types=[
          pltpu.VMEM_SHARED((n_pad,), jnp.float32),
          pltpu.VMEM((CH,), jnp.int32),
          pltpu.VMEM((CH,), jnp.float32),
      ],
  )
  def deg_kernel(dst_hbm, zeros_hbm, d0_hbm, d1_hbm, deg_sh, idx_v, ones_v):
    c = lax.axis_index("c")
    s = lax.axis_index("s")
    wid = c * NS + s
    for j in range(CH // 16):
      ones_v[pl.ds(j * 16, 16)] = jnp.ones((16,), jnp.float32)
    row0 = s * rows_per_s
    pltpu.sync_copy(zeros_hbm.at[pl.ds(row0, rows_per_s)],
                    deg_sh.at[pl.ds(row0, rows_per_s)])
    plsc.subcore_barrier()
    base = wid * per_w

    def body(i, carry):
      pltpu.sync_copy(dst_hbm.at[pl.ds(base + i * CH, CH)], idx_v)
      pltpu.sync_copy(ones_v, deg_sh.at[idx_v], add=True)
      return carry

    lax.fori_loop(0, n_chunks, body, 0)
    plsc.subcore_barrier()

    @pl.when(c == 0)
    def _():
      pltpu.sync_copy(deg_sh.at[pl.ds(row0, rows_per_s)],
                      d0_hbm.at[pl.ds(row0, rows_per_s)])

    @pl.when(c == 1)
    def _():
      pltpu.sync_copy(deg_sh.at[pl.ds(row0, rows_per_s)],
                      d1_hbm.at[pl.ds(row0, rows_per_s)])

  return deg_kernel


def _make_edge_kernel(n_pad, e_pad, d):
  per_w = e_pad // NW
  n_chunks = per_w // CH
  rows_per_s = n_pad // NS

  @functools.partial(
      pl.kernel,
      out_type=(jax.ShapeDtypeStruct((n_pad, d), jnp.float32),
                jax.ShapeDtypeStruct((n_pad, d), jnp.float32)),
      mesh=_sc_mesh(),
      scratch_types=[
          pltpu.VMEM_SHARED((n_pad, d), jnp.float32),
          pltpu.VMEM((CH,), jnp.int32),
          pltpu.VMEM((CH,), jnp.int32),
          pltpu.VMEM((CH, d), jnp.float32),
          pltpu.SemaphoreType.DMA,
      ],
  )
  def edge_kernel(h2_hbm, src_hbm, dst_hbm, p0_hbm, p1_hbm,
                  acc_sh, src_v, dst_v, rows_v, sem):
    c = lax.axis_index("c")
    s = lax.axis_index("s")
    wid = c * NS + s
    row0 = s * rows_per_s
    pltpu.sync_copy(h2_hbm.at[pl.ds(row0, rows_per_s)],
                    acc_sh.at[pl.ds(row0, rows_per_s)])
    plsc.subcore_barrier()
    base = wid * per_w

    def body(i, carry):
      off = base + i * CH
      pltpu.sync_copy(src_hbm.at[pl.ds(off, CH)], src_v)
      pltpu.sync_copy(dst_hbm.at[pl.ds(off, CH)], dst_v)
      pltpu.async_copy(h2_hbm.at[src_v], rows_v, sem).wait()
      pltpu.sync_copy(rows_v, acc_sh.at[dst_v], add=True)
      return carry

    lax.fori_loop(0, n_chunks, body, 0)
    plsc.subcore_barrier()

    @pl.when(c == 0)
    def _():
      pltpu.sync_copy(acc_sh.at[pl.ds(row0, rows_per_s)],
                      p0_hbm.at[pl.ds(row0, rows_per_s)])

    @pl.when(c == 1)
    def _():
      pltpu.sync_copy(acc_sh.at[pl.ds(row0, rows_per_s)],
                      p1_hbm.at[pl.ds(row0, rows_per_s)])

  return edge_kernel


def _prescale_body(x_ref, w_ref, d0_ref, d1_ref, h2_ref):
  dis = lax.rsqrt(1.0 + d0_ref[...] + d1_ref[...])
  h2_ref[...] = jnp.dot(x_ref[...] * dis, w_ref[...],
                        preferred_element_type=jnp.float32)


def _finalize_body(p0_ref, p1_ref, h2_ref, x_ref, d0_ref, d1_ref,
                   b_ref, g_ref, be_ref, o_ref):
  dis = lax.rsqrt(1.0 + d0_ref[...] + d1_ref[...])
  acc = p0_ref[...] + p1_ref[...] - h2_ref[...]
  out = acc * dis + b_ref[...]
  mu = jnp.mean(out, axis=-1, keepdims=True)
  var = jnp.mean((out - mu) ** 2, axis=-1, keepdims=True)
  ln = (out - mu) * lax.rsqrt(var + 1e-5) * g_ref[...] + be_ref[...]
  o_ref[...] = jnp.maximum(ln + x_ref[...], 0.0)


def kernel(x, edge_index, W, b, gamma, beta):
  n, d = x.shape
  e = edge_index.shape[1]
  n_pad = ((n + BLK - 1) // BLK) * BLK
  e_pad = ((e + NW * CH - 1) // (NW * CH)) * (NW * CH)

  src = edge_index[0].astype(jnp.int32)
  dst = edge_index[1].astype(jnp.int32)
  pad_idx = jnp.full((e_pad - e,), n_pad - 1, dtype=jnp.int32)
  src = jnp.concatenate([src, pad_idx])
  dst = jnp.concatenate([dst, pad_idx])
  x_pad = jnp.concatenate([x, jnp.zeros((n_pad - n, d), jnp.float32)])

  d0, d1 = _make_deg_kernel(n_pad, e_pad)(dst, jnp.zeros((n_pad,), jnp.float32))
  d0c = d0.reshape(n_pad, 1)
  d1c = d1.reshape(n_pad, 1)

  grid = (n_pad // BLK,)
  row_spec = pl.BlockSpec((BLK, d), lambda i: (i, 0))
  col_spec = pl.BlockSpec((BLK, 1), lambda i: (i, 0))
  vec_spec = pl.BlockSpec((1, d), lambda i: (0, 0))

  h2 = pl.pallas_call(
      _prescale_body,
      grid=grid,
      in_specs=[row_spec, pl.BlockSpec((d, d), lambda i: (0, 0)),
                col_spec, col_spec],
      out_specs=row_spec,
      out_shape=jax.ShapeDtypeStruct((n_pad, d), jnp.float32),
  )(x_pad, W, d0c, d1c)

  p0, p1 = _make_edge_kernel(n_pad, e_pad, d)(h2, src, dst)

  out_pad = pl.pallas_call(
      _finalize_body,
      grid=grid,
      in_specs=[row_spec, row_spec, row_spec, row_spec, col_spec, col_spec,
                vec_spec, vec_spec, vec_spec],
      out_specs=row_spec,
      out_shape=jax.ShapeDtypeStruct((n_pad, d), jnp.float32),
  )(p0, p1, h2, x_pad, d0c, d1c,
    b.reshape(1, d), gamma.reshape(1, d), beta.reshape(1, d))

  return out_pad[:n]


# speedup vs baseline: 36.3556x; 2.6563x over previous
import functools

import jax
import jax.numpy as jnp
from jax import lax
from jax.experimental import pallas as pl
from jax.experimental.pallas import tpu as pltpu
from jax.experimental.pallas import tpu_sc as plsc

NC = 2
NS = 16
NW = NC * NS
DCH = 128
CH = 64
NBUF = 4
BLK = 1280


def _sc_mesh():
  return plsc.VectorSubcoreMesh(
      core_axis_name="c", subcore_axis_name="s",
      num_cores=NC, num_subcores=NS)


def _make_deg_kernel(n_pad, e_pad):
  per_w = e_pad // NW
  n_chunks = per_w // DCH
  rows_per_s = n_pad // NS

  @functools.partial(
      pl.kernel,
      out_type=(jax.ShapeDtypeStruct((n_pad,), jnp.float32),
                jax.ShapeDtypeStruct((n_pad,), jnp.float32)),
      mesh=_sc_mesh(),
      scratch_types=[
          pltpu.VMEM_SHARED((n_pad,), jnp.float32),
          pltpu.VMEM((n_chunks, DCH), jnp.int32),
          pltpu.VMEM((DCH,), jnp.float32),
          pltpu.SemaphoreType.DMA,
      ],
  )
  def deg_kernel(dst_hbm, zeros_hbm, d0_hbm, d1_hbm,
                 deg_sh, dsts_v, ones_v, sem):
    c = lax.axis_index("c")
    sax = lax.axis_index("s")
    wid = c * NS + sax
    for j in range(DCH // 16):
      ones_v[pl.ds(j * 16, 16)] = jnp.ones((16,), jnp.float32)
    row0 = sax * rows_per_s
    pltpu.sync_copy(zeros_hbm.at[pl.ds(row0, rows_per_s)],
                    deg_sh.at[pl.ds(row0, rows_per_s)])
    pltpu.sync_copy(dst_hbm.at[wid], dsts_v)
    plsc.subcore_barrier()

    def fire(i, carry):
      pltpu.async_copy(ones_v, deg_sh.at[dsts_v.at[i]], sem, add=True)
      return carry

    lax.fori_loop(0, n_chunks, fire, 0)

    def drain(i, carry):
      pltpu.make_async_copy(zeros_hbm.at[pl.ds(0, DCH)], ones_v, sem).wait()
      return carry

    lax.fori_loop(0, n_chunks, drain, 0)
    plsc.subcore_barrier()

    @pl.when(c == 0)
    def _():
      pltpu.sync_copy(deg_sh.at[pl.ds(row0, rows_per_s)],
                      d0_hbm.at[pl.ds(row0, rows_per_s)])

    @pl.when(c == 1)
    def _():
      pltpu.sync_copy(deg_sh.at[pl.ds(row0, rows_per_s)],
                      d1_hbm.at[pl.ds(row0, rows_per_s)])

  return deg_kernel


def _make_edge_kernel(n_pad, e_pad, d):
  per_w = e_pad // NW
  n_chunks = per_w // CH
  n_rounds = n_chunks // NBUF
  rows_per_s = n_pad // NS

  @functools.partial(
      pl.kernel,
      out_type=(jax.ShapeDtypeStruct((n_pad, d), jnp.float32),
                jax.ShapeDtypeStruct((n_pad, d), jnp.float32)),
      mesh=_sc_mesh(),
      scratch_types=[
          pltpu.VMEM_SHARED((n_pad, d), jnp.float32),
          pltpu.VMEM((per_w,), jnp.int32),
      ] + [pltpu.VMEM((CH,), jnp.int32)] * NBUF
        + [pltpu.VMEM((CH, d), jnp.float32)] * NBUF
        + [pltpu.SemaphoreType.DMA] * (3 * NBUF),
  )
  def edge_kernel(h2_hbm, src_hbm, dst_hbm, p0_hbm, p1_hbm,
                  acc_sh, srcs_v, *bufs_and_sems):
    dstb = bufs_and_sems[:NBUF]
    rows = bufs_and_sems[NBUF:2 * NBUF]
    gsem = bufs_and_sems[2 * NBUF:3 * NBUF]
    ssem = bufs_and_sems[3 * NBUF:4 * NBUF]
    dsem = bufs_and_sems[4 * NBUF:]
    c = lax.axis_index("c")
    sax = lax.axis_index("s")
    wid = c * NS + sax
    row0 = sax * rows_per_s
    pltpu.sync_copy(h2_hbm.at[pl.ds(row0, rows_per_s)],
                    acc_sh.at[pl.ds(row0, rows_per_s)])
    pltpu.sync_copy(src_hbm.at[wid], srcs_v)
    plsc.subcore_barrier()

    dst_base = wid * n_chunks
    for b in range(NBUF):
      pltpu.async_copy(dst_hbm.at[dst_base + b], dstb[b], dsem[b])
      pltpu.async_copy(h2_hbm.at[srcs_v.at[pl.ds(b * CH, CH)]],
                       rows[b], gsem[b])

    def round_body(j, carry):
      for b in range(NBUF):
        pltpu.make_async_copy(dst_hbm.at[0], dstb[b], dsem[b]).wait()
        pltpu.make_async_copy(h2_hbm.at[pl.ds(0, CH)], rows[b],
                              gsem[b]).wait()
        pltpu.async_copy(rows[b], acc_sh.at[dstb[b]], ssem[b], add=True)
      for b in range(NBUF):
        nxt = NBUF * (j + 1) + b
        pltpu.make_async_copy(h2_hbm.at[pl.ds(0, CH)], rows[b],
                              ssem[b]).wait()

        @pl.when(nxt < n_chunks)
        def _():
          pltpu.async_copy(dst_hbm.at[dst_base + nxt], dstb[b], dsem[b])
          pltpu.async_copy(h2_hbm.at[srcs_v.at[pl.ds(nxt * CH, CH)]],
                           rows[b], gsem[b])

      return carry

    lax.fori_loop(0, n_rounds, round_body, 0)
    plsc.subcore_barrier()

    @pl.when(c == 0)
    def _():
      pltpu.sync_copy(acc_sh.at[pl.ds(row0, rows_per_s)],
                      p0_hbm.at[pl.ds(row0, rows_per_s)])

    @pl.when(c == 1)
    def _():
      pltpu.sync_copy(acc_sh.at[pl.ds(row0, rows_per_s)],
                      p1_hbm.at[pl.ds(row0, rows_per_s)])

  return edge_kernel


def _prescale_body(x_ref, w_ref, d0_ref, d1_ref, h2_ref):
  dis = lax.rsqrt(1.0 + d0_ref[...] + d1_ref[...])
  h2_ref[...] = jnp.dot(x_ref[...] * dis, w_ref[...],
                        preferred_element_type=jnp.float32)


def _finalize_body(p0_ref, p1_ref, h2_ref, x_ref, d0_ref, d1_ref,
                   b_ref, g_ref, be_ref, o_ref):
  dis = lax.rsqrt(1.0 + d0_ref[...] + d1_ref[...])
  acc = p0_ref[...] + p1_ref[...] - h2_ref[...]
  out = acc * dis + b_ref[...]
  mu = jnp.mean(out, axis=-1, keepdims=True)
  var = jnp.mean((out - mu) ** 2, axis=-1, keepdims=True)
  ln = (out - mu) * lax.rsqrt(var + 1e-5) * g_ref[...] + be_ref[...]
  o_ref[...] = jnp.maximum(ln + x_ref[...], 0.0)


def kernel(x, edge_index, W, b, gamma, beta):
  n, d = x.shape
  e = edge_index.shape[1]
  n_pad = ((n + BLK) // BLK) * BLK
  egrain = NW * max(CH * NBUF, DCH)
  e_pad = ((e + egrain - 1) // egrain) * egrain
  per_w = e_pad // NW
  n_chunks = per_w // CH

  src = edge_index[0].astype(jnp.int32)
  dst = edge_index[1].astype(jnp.int32)
  pad_idx = n + (jnp.arange(e_pad - e, dtype=jnp.int32) % (n_pad - n))
  src2 = jnp.concatenate([src, pad_idx]).reshape(NW, per_w)
  dst_flat = jnp.concatenate([dst, pad_idx])
  dst2 = dst_flat.reshape(NW * n_chunks, CH)
  dst_deg = dst_flat.reshape(NW, per_w // DCH, DCH)
  x_pad = jnp.concatenate([x, jnp.zeros((n_pad - n, d), jnp.float32)])

  d0, d1 = _make_deg_kernel(n_pad, e_pad)(
      dst_deg, jnp.zeros((n_pad,), jnp.float32))
  d0c = d0.reshape(n_pad, 1)
  d1c = d1.reshape(n_pad, 1)

  grid = (n_pad // BLK,)
  row_spec = pl.BlockSpec((BLK, d), lambda i: (i, 0))
  col_spec = pl.BlockSpec((BLK, 1), lambda i: (i, 0))
  vec_spec = pl.BlockSpec((1, d), lambda i: (0, 0))

  h2 = pl.pallas_call(
      _prescale_body,
      grid=grid,
      in_specs=[row_spec, pl.BlockSpec((d, d), lambda i: (0, 0)),
                col_spec, col_spec],
      out_specs=row_spec,
      out_shape=jax.ShapeDtypeStruct((n_pad, d), jnp.float32),
  )(x_pad, W, d0c, d1c)

  p0, p1 = _make_edge_kernel(n_pad, e_pad, d)(h2, src2, dst2)

  out_pad = pl.pallas_call(
      _finalize_body,
      grid=grid,
      in_specs=[row_spec, row_spec, row_spec, row_spec, col_spec, col_spec,
                vec_spec, vec_spec, vec_spec],
      out_specs=row_spec,
      out_shape=jax.ShapeDtypeStruct((n_pad, d), jnp.float32),
  )(p0, p1, h2, x_pad, d0c, d1c,
    b.reshape(1, d), gamma.reshape(1, d), beta.reshape(1, d))

  return out_pad[:n]
